# baseline (speedup 1.0000x reference)
import functools
import math

import jax
import jax.numpy as jnp
from jax import lax
from jax.experimental import pallas as pl
from jax.experimental.pallas import tpu as pltpu
from jax.experimental.pallas import tpu_sc as plsc

TBLK = 1024


def _pack_bf16(a, b):
    au = lax.bitcast_convert_type(a, jnp.uint32)
    bu = lax.bitcast_convert_type(b, jnp.uint32)
    packed = ((au + 0x8000) >> 16) | ((bu + 0x8000) & jnp.uint32(0xFFFF0000))
    return lax.bitcast_convert_type(packed, jnp.int32)


def _unpack_bf16(p):
    lo = lax.bitcast_convert_type(p << 16, jnp.float32)
    hi = lax.bitcast_convert_type(p & jnp.int32(-65536), jnp.float32)
    return lo, hi


def _gate_body(C, E, x_ref, wg_ref, sidx_ref, gidx_ref, g_ref, xpack_ref,
               counts_ref, carry_ref, e_stash_ref, g_stash_ref):
    j = pl.program_id(0)
    b = pl.program_id(1)
    T = e_stash_ref.shape[0] // 2
    D2 = xpack_ref.shape[1]

    @pl.when(jnp.logical_and(j == 0, b == 0))
    def _():
        carry_ref[...] = jnp.zeros_like(carry_ref)

    xb = x_ref[...]
    xpack_ref[...] = _pack_bf16(xb[:, :D2], xb[:, D2:])

    @pl.when(j == 0)
    def _():
        logits = jnp.dot(xb, wg_ref[...], preferred_element_type=jnp.float32)
        m = jnp.max(logits, axis=1, keepdims=True)
        p = jnp.exp(logits - m)
        probs = p / jnp.sum(p, axis=1, keepdims=True)

        lane = lax.broadcasted_iota(jnp.int32, (TBLK, E), 1)
        v1 = jnp.max(probs, axis=1, keepdims=True)
        i1 = jnp.min(jnp.where(probs == v1, lane, E), axis=1, keepdims=True)
        probs2 = jnp.where(lane == i1, -1.0, probs)
        v2 = jnp.max(probs2, axis=1, keepdims=True)
        i2 = jnp.min(jnp.where(probs2 == v2, lane, E), axis=1, keepdims=True)
        ssum = v1 + v2
        e_stash_ref[pl.ds(b * TBLK, TBLK), :] = i1
        g_stash_ref[pl.ds(b * TBLK, TBLK), :] = v1 / ssum
        e_stash_ref[pl.ds(T + b * TBLK, TBLK), :] = i2
        g_stash_ref[pl.ds(T + b * TBLK, TBLK), :] = v2 / ssum

    e_t = e_stash_ref[pl.ds(j * T + b * TBLK, TBLK), :]
    g_t = g_stash_ref[pl.ds(j * T + b * TBLK, TBLK), :]

    lane = lax.broadcasted_iota(jnp.int32, (TBLK, E), 1)
    onehot = (lane == e_t).astype(jnp.float32)
    r0 = lax.broadcasted_iota(jnp.int32, (TBLK, TBLK), 0)
    r1 = lax.broadcasted_iota(jnp.int32, (TBLK, TBLK), 1)
    tri = (r1 <= r0).astype(jnp.float32)
    incl = jnp.dot(tri, onehot, preferred_element_type=jnp.float32)
    carry_old = carry_ref[...]
    pos_mat = incl - onehot + carry_old
    pos_t = jnp.sum(pos_mat * onehot, axis=1, keepdims=True)
    valid = pos_t < float(C)
    carry_ref[...] = carry_old + jnp.sum(
        onehot * valid.astype(jnp.float32), axis=0, keepdims=True)

    pos_i = pos_t.astype(jnp.int32)
    lidx = e_t * C + pos_i
    sidx_ref[...] = jnp.where(valid, lidx, E * C).reshape(1, TBLK, 1)
    gidx_ref[...] = jnp.where(valid, lidx, 0).reshape(1, TBLK, 1)
    g_ref[...] = jnp.where(valid, g_t, 0.0).reshape(1, TBLK, 1)
    counts_ref[...] = carry_ref[...].astype(jnp.int32)


def _gate(x, wg, C, K):
    T, D = x.shape
    E = wg.shape[1]
    NB = T // TBLK
    sidx3, gidx3, gates, xpack, counts2 = pl.pallas_call(
        functools.partial(_gate_body, C, E),
        grid=(K, NB),
        in_specs=[
            pl.BlockSpec((TBLK, D), lambda j, b: (b, 0)),
            pl.BlockSpec((D, E), lambda j, b: (0, 0)),
        ],
        out_specs=[
            pl.BlockSpec((1, TBLK, 1), lambda j, b: (j * NB + b, 0, 0)),
            pl.BlockSpec((1, TBLK, 1), lambda j, b: (j * NB + b, 0, 0)),
            pl.BlockSpec((1, TBLK, 1), lambda j, b: (j * NB + b, 0, 0)),
            pl.BlockSpec((TBLK, D // 2), lambda j, b: (b, 0)),
            pl.BlockSpec((1, E), lambda j, b: (0, 0)),
        ],
        out_shape=[
            jax.ShapeDtypeStruct((K * NB, TBLK, 1), jnp.int32),
            jax.ShapeDtypeStruct((K * NB, TBLK, 1), jnp.int32),
            jax.ShapeDtypeStruct((K * NB, TBLK, 1), jnp.float32),
            jax.ShapeDtypeStruct((T, D // 2), jnp.int32),
            jax.ShapeDtypeStruct((1, E), jnp.int32),
        ],
        scratch_shapes=[
            pltpu.VMEM((1, E), jnp.float32),
            pltpu.VMEM((2 * T, 1), jnp.int32),
            pltpu.VMEM((2 * T, 1), jnp.float32),
        ],
        compiler_params=pltpu.CompilerParams(
            dimension_semantics=("arbitrary", "arbitrary")),
    )(x, wg)
    return (sidx3.reshape(K * T), gidx3.reshape(K * T),
            gates.reshape(K * T, 1), xpack, counts2.reshape(E))


def _sc_dispatch(x, sidx, n_rows):
    T, D = x.shape
    B = sidx.shape[0]
    NW = 32
    b_per_w = B // NW
    mesh = plsc.VectorSubcoreMesh(core_axis_name="c", subcore_axis_name="s")

    @functools.partial(
        pl.kernel,
        mesh=mesh,
        out_type=jax.ShapeDtypeStruct((n_rows, D), x.dtype),
        scratch_types=[
            pltpu.VMEM((b_per_w,), jnp.int32),
            pltpu.VMEM((b_per_w, D), x.dtype),
            pltpu.SemaphoreType.DMA,
        ],
    )
    def k(x_hbm, sidx_hbm, xe_hbm, idx_v, rows_v, sem):
        wid = lax.axis_index("s") * 2 + lax.axis_index("c")
        base = wid * b_per_w
        pltpu.sync_copy(sidx_hbm.at[pl.ds(base, b_per_w)], idx_v)
        tbase = lax.rem(base, T)
        pltpu.sync_copy(x_hbm.at[pl.ds(tbase, b_per_w)], rows_v)
        pltpu.async_copy(rows_v, xe_hbm.at[idx_v], sem).wait()

    return k(x, sidx)


def _sc_gather(ye, gidx):
    D = ye.shape[1]
    B = gidx.shape[0]
    NW = 32
    b_per_w = B // NW
    mesh = plsc.VectorSubcoreMesh(core_axis_name="c", subcore_axis_name="s")

    @functools.partial(
        pl.kernel,
        mesh=mesh,
        out_type=jax.ShapeDtypeStruct((B, D), ye.dtype),
        scratch_types=[
            pltpu.VMEM((b_per_w,), jnp.int32),
            pltpu.VMEM((b_per_w, D), ye.dtype),
            pltpu.SemaphoreType.DMA,
        ],
    )
    def k(ye_hbm, gidx_hbm, out_hbm, idx_v, rows_v, sem):
        wid = lax.axis_index("s") * 2 + lax.axis_index("c")
        base = wid * b_per_w
        pltpu.sync_copy(gidx_hbm.at[pl.ds(base, b_per_w)], idx_v)
        pltpu.async_copy(ye_hbm.at[idx_v], rows_v, sem).wait()
        pltpu.sync_copy(rows_v, out_hbm.at[pl.ds(base, b_per_w)])

    return k(ye, gidx)


def _ffn_body(C, EG, counts_sref, xe_ref, w1_ref, b1_ref, w2_ref, b2_ref,
              ye_ref):
    e = pl.program_id(0)
    row = lax.broadcasted_iota(jnp.int32, (C, 1), 0)
    D2 = ye_ref.shape[1]
    for k in range(EG):
        cnt = counts_sref[e * EG + k]
        xlo, xhi = _unpack_bf16(xe_ref[pl.ds(k * C, C), :])
        xb = jnp.concatenate([xlo, xhi], axis=1)
        xb = jnp.where(row < cnt, xb, 0.0)
        h = (jnp.dot(xb, w1_ref[k], preferred_element_type=jnp.float32) +
             b1_ref[k])
        h = 0.5 * h * (1.0 + lax.erf(h * (1.0 / math.sqrt(2.0))))
        y = (jnp.dot(h, w2_ref[k], preferred_element_type=jnp.float32) +
             b2_ref[k])
        ye_ref[pl.ds(k * C, C), :] = _pack_bf16(y[:, :D2], y[:, D2:])


def _ffn(counts, xe, W1, b1, W2, b2, C, EG=1):
    E, D, F = W1.shape
    grid_spec = pltpu.PrefetchScalarGridSpec(
        num_scalar_prefetch=1,
        grid=(E // EG,),
        in_specs=[
            pl.BlockSpec((EG * C, D // 2), lambda e, counts: (e, 0)),
            pl.BlockSpec((EG, D, F), lambda e, counts: (e, 0, 0)),
            pl.BlockSpec((EG, 1, F), lambda e, counts: (e, 0, 0)),
            pl.BlockSpec((EG, F, D), lambda e, counts: (e, 0, 0)),
            pl.BlockSpec((EG, 1, D), lambda e, counts: (e, 0, 0)),
        ],
        out_specs=pl.BlockSpec((EG * C, D // 2), lambda e, counts: (e, 0)),
    )
    return pl.pallas_call(
        functools.partial(_ffn_body, C, EG),
        grid_spec=grid_spec,
        out_shape=jax.ShapeDtypeStruct((E * C, D // 2), jnp.int32),
        compiler_params=pltpu.CompilerParams(
            dimension_semantics=("arbitrary",)),
    )(counts, xe, W1, b1.reshape(E, 1, F), W2, b2.reshape(E, 1, D))


def _wadd_body(ya_ref, yb_ref, g0_ref, g1_ref, y_ref):
    alo, ahi = _unpack_bf16(ya_ref[...])
    blo, bhi = _unpack_bf16(yb_ref[...])
    g0 = g0_ref[...]
    g1 = g1_ref[...]
    y_ref[...] = jnp.concatenate(
        [g0 * alo + g1 * blo, g0 * ahi + g1 * bhi], axis=1)


def _wadd(yab, gates, T, D):
    WB = 512
    NB = T // WB
    return pl.pallas_call(
        _wadd_body,
        grid=(NB,),
        in_specs=[
            pl.BlockSpec((WB, D // 2), lambda b: (b, 0)),
            pl.BlockSpec((WB, D // 2), lambda b: (NB + b, 0)),
            pl.BlockSpec((WB, 1), lambda b: (b, 0)),
            pl.BlockSpec((WB, 1), lambda b: (NB + b, 0)),
        ],
        out_specs=pl.BlockSpec((WB, D), lambda b: (b, 0)),
        out_shape=jax.ShapeDtypeStruct((T, D), jnp.float32),
    )(yab, yab, gates, gates)


def kernel(x, wg, W1, b1, W2, b2):
    T, D = x.shape
    E, _, F = W1.shape
    K = 2
    C = int(math.ceil(K * T / E * 1.25))

    sidx, gidx, gates, xpack, counts = _gate(x, wg, C, K)
    xe = _sc_dispatch(xpack, sidx, (E + 2) * C)
    ye = _ffn(counts, xe, W1, b1, W2, b2, C)
    yab = _sc_gather(ye, gidx)
    return _wadd(yab, gates, T, D)


# speedup vs baseline: 1.0120x; 1.0120x over previous
import functools
import math

import jax
import jax.numpy as jnp
from jax import lax
from jax.experimental import pallas as pl
from jax.experimental.pallas import tpu as pltpu
from jax.experimental.pallas import tpu_sc as plsc

TBLK = 1024


def _pack_bf16(a, b):
    au = lax.bitcast_convert_type(a, jnp.uint32)
    bu = lax.bitcast_convert_type(b, jnp.uint32)
    packed = ((au + 0x8000) >> 16) | ((bu + 0x8000) & jnp.uint32(0xFFFF0000))
    return lax.bitcast_convert_type(packed, jnp.int32)


def _unpack_bf16(p):
    lo = lax.bitcast_convert_type(p << 16, jnp.float32)
    hi = lax.bitcast_convert_type(p & jnp.int32(-65536), jnp.float32)
    return lo, hi


def _gate_body(C, E, x_ref, wg_ref, sidx_ref, gidx_ref, g_ref, xpack_ref,
               counts_ref, carry_ref, e_stash_ref, g_stash_ref, tri_ref):
    j = pl.program_id(0)
    b = pl.program_id(1)
    T = e_stash_ref.shape[0] // 2
    D2 = xpack_ref.shape[1]

    @pl.when(jnp.logical_and(j == 0, b == 0))
    def _():
        carry_ref[...] = jnp.zeros_like(carry_ref)
        r0 = lax.broadcasted_iota(jnp.int32, (TBLK, TBLK), 0)
        r1 = lax.broadcasted_iota(jnp.int32, (TBLK, TBLK), 1)
        tri_ref[...] = (r1 <= r0).astype(jnp.float32)

    xb = x_ref[...]
    xpack_ref[...] = _pack_bf16(xb[:, :D2], xb[:, D2:])

    @pl.when(j == 0)
    def _():
        logits = jnp.dot(xb, wg_ref[...], preferred_element_type=jnp.float32)
        m = jnp.max(logits, axis=1, keepdims=True)
        p = jnp.exp(logits - m)
        probs = p / jnp.sum(p, axis=1, keepdims=True)

        lane = lax.broadcasted_iota(jnp.int32, (TBLK, E), 1)
        v1 = jnp.max(probs, axis=1, keepdims=True)
        i1 = jnp.min(jnp.where(probs == v1, lane, E), axis=1, keepdims=True)
        probs2 = jnp.where(lane == i1, -1.0, probs)
        v2 = jnp.max(probs2, axis=1, keepdims=True)
        i2 = jnp.min(jnp.where(probs2 == v2, lane, E), axis=1, keepdims=True)
        ssum = v1 + v2
        e_stash_ref[pl.ds(b * TBLK, TBLK), :] = i1
        g_stash_ref[pl.ds(b * TBLK, TBLK), :] = v1 / ssum
        e_stash_ref[pl.ds(T + b * TBLK, TBLK), :] = i2
        g_stash_ref[pl.ds(T + b * TBLK, TBLK), :] = v2 / ssum

    e_t = e_stash_ref[pl.ds(j * T + b * TBLK, TBLK), :]
    g_t = g_stash_ref[pl.ds(j * T + b * TBLK, TBLK), :]

    lane = lax.broadcasted_iota(jnp.int32, (TBLK, E), 1)
    onehot = (lane == e_t).astype(jnp.float32)
    incl = jnp.dot(tri_ref[...], onehot, preferred_element_type=jnp.float32)
    carry_old = carry_ref[...]
    pos_mat = incl - onehot + carry_old
    pos_t = jnp.sum(pos_mat * onehot, axis=1, keepdims=True)
    valid = pos_t < float(C)
    carry_ref[...] = carry_old + jnp.sum(
        onehot * valid.astype(jnp.float32), axis=0, keepdims=True)

    pos_i = pos_t.astype(jnp.int32)
    lidx = e_t * C + pos_i
    sidx_ref[...] = jnp.where(valid, lidx, E * C).reshape(1, TBLK, 1)
    gidx_ref[...] = jnp.where(valid, lidx, 0).reshape(1, TBLK, 1)
    g_ref[...] = jnp.where(valid, g_t, 0.0).reshape(1, TBLK, 1)
    counts_ref[...] = carry_ref[...].astype(jnp.int32)


def _gate(x, wg, C, K):
    T, D = x.shape
    E = wg.shape[1]
    NB = T // TBLK
    sidx3, gidx3, gates, xpack, counts2 = pl.pallas_call(
        functools.partial(_gate_body, C, E),
        grid=(K, NB),
        in_specs=[
            pl.BlockSpec((TBLK, D), lambda j, b: (b, 0)),
            pl.BlockSpec((D, E), lambda j, b: (0, 0)),
        ],
        out_specs=[
            pl.BlockSpec((1, TBLK, 1), lambda j, b: (j * NB + b, 0, 0)),
            pl.BlockSpec((1, TBLK, 1), lambda j, b: (j * NB + b, 0, 0)),
            pl.BlockSpec((1, TBLK, 1), lambda j, b: (j * NB + b, 0, 0)),
            pl.BlockSpec((TBLK, D // 2), lambda j, b: (b, 0)),
            pl.BlockSpec((1, E), lambda j, b: (0, 0)),
        ],
        out_shape=[
            jax.ShapeDtypeStruct((K * NB, TBLK, 1), jnp.int32),
            jax.ShapeDtypeStruct((K * NB, TBLK, 1), jnp.int32),
            jax.ShapeDtypeStruct((K * NB, TBLK, 1), jnp.float32),
            jax.ShapeDtypeStruct((T, D // 2), jnp.int32),
            jax.ShapeDtypeStruct((1, E), jnp.int32),
        ],
        scratch_shapes=[
            pltpu.VMEM((1, E), jnp.float32),
            pltpu.VMEM((2 * T, 1), jnp.int32),
            pltpu.VMEM((2 * T, 1), jnp.float32),
            pltpu.VMEM((TBLK, TBLK), jnp.float32),
        ],
        compiler_params=pltpu.CompilerParams(
            dimension_semantics=("arbitrary", "arbitrary")),
    )(x, wg)
    return (sidx3.reshape(K * T), gidx3.reshape(K * T),
            gates.reshape(K * T, 1), xpack, counts2.reshape(E))


def _sc_dispatch(x, sidx, n_rows):
    T, D = x.shape
    B = sidx.shape[0]
    NW = 32
    b_per_w = B // NW
    mesh = plsc.VectorSubcoreMesh(core_axis_name="c", subcore_axis_name="s")

    @functools.partial(
        pl.kernel,
        mesh=mesh,
        out_type=jax.ShapeDtypeStruct((n_rows, D), x.dtype),
        scratch_types=[
            pltpu.VMEM((b_per_w,), jnp.int32),
            pltpu.VMEM((b_per_w, D), x.dtype),
            pltpu.SemaphoreType.DMA,
        ],
    )
    def k(x_hbm, sidx_hbm, xe_hbm, idx_v, rows_v, sem):
        wid = lax.axis_index("s") * 2 + lax.axis_index("c")
        base = wid * b_per_w
        pltpu.sync_copy(sidx_hbm.at[pl.ds(base, b_per_w)], idx_v)
        tbase = lax.rem(base, T)
        pltpu.sync_copy(x_hbm.at[pl.ds(tbase, b_per_w)], rows_v)
        pltpu.async_copy(rows_v, xe_hbm.at[idx_v], sem).wait()

    return k(x, sidx)


def _sc_gather(ye, gidx):
    D = ye.shape[1]
    B = gidx.shape[0]
    NW = 32
    b_per_w = B // NW
    mesh = plsc.VectorSubcoreMesh(core_axis_name="c", subcore_axis_name="s")

    @functools.partial(
        pl.kernel,
        mesh=mesh,
        out_type=jax.ShapeDtypeStruct((B, D), ye.dtype),
        scratch_types=[
            pltpu.VMEM((b_per_w,), jnp.int32),
            pltpu.VMEM((b_per_w, D), ye.dtype),
            pltpu.SemaphoreType.DMA,
        ],
    )
    def k(ye_hbm, gidx_hbm, out_hbm, idx_v, rows_v, sem):
        wid = lax.axis_index("s") * 2 + lax.axis_index("c")
        base = wid * b_per_w
        pltpu.sync_copy(gidx_hbm.at[pl.ds(base, b_per_w)], idx_v)
        pltpu.async_copy(ye_hbm.at[idx_v], rows_v, sem).wait()
        pltpu.sync_copy(rows_v, out_hbm.at[pl.ds(base, b_per_w)])

    return k(ye, gidx)


def _ffn_body(C, EG, counts_sref, xe_ref, w1_ref, b1_ref, w2_ref, b2_ref,
              ye_ref):
    e = pl.program_id(0)
    row = lax.broadcasted_iota(jnp.int32, (C, 1), 0)
    D2 = ye_ref.shape[1]
    for k in range(EG):
        cnt = counts_sref[e * EG + k]
        xlo, xhi = _unpack_bf16(xe_ref[pl.ds(k * C, C), :])
        xb = jnp.concatenate([xlo, xhi], axis=1)
        xb = jnp.where(row < cnt, xb, 0.0)
        h = (jnp.dot(xb, w1_ref[k], preferred_element_type=jnp.float32) +
             b1_ref[k])
        h = 0.5 * h * (1.0 + lax.erf(h * (1.0 / math.sqrt(2.0))))
        y = (jnp.dot(h, w2_ref[k], preferred_element_type=jnp.float32) +
             b2_ref[k])
        ye_ref[pl.ds(k * C, C), :] = _pack_bf16(y[:, :D2], y[:, D2:])


def _ffn(counts, xe, W1, b1, W2, b2, C, EG=1):
    E, D, F = W1.shape
    grid_spec = pltpu.PrefetchScalarGridSpec(
        num_scalar_prefetch=1,
        grid=(E // EG,),
        in_specs=[
            pl.BlockSpec((EG * C, D // 2), lambda e, counts: (e, 0)),
            pl.BlockSpec((EG, D, F), lambda e, counts: (e, 0, 0)),
            pl.BlockSpec((EG, 1, F), lambda e, counts: (e, 0, 0)),
            pl.BlockSpec((EG, F, D), lambda e, counts: (e, 0, 0)),
            pl.BlockSpec((EG, 1, D), lambda e, counts: (e, 0, 0)),
        ],
        out_specs=pl.BlockSpec((EG * C, D // 2), lambda e, counts: (e, 0)),
    )
    return pl.pallas_call(
        functools.partial(_ffn_body, C, EG),
        grid_spec=grid_spec,
        out_shape=jax.ShapeDtypeStruct((E * C, D // 2), jnp.int32),
        compiler_params=pltpu.CompilerParams(
            dimension_semantics=("arbitrary",)),
    )(counts, xe, W1, b1.reshape(E, 1, F), W2, b2.reshape(E, 1, D))


def _wadd_body(ya_ref, yb_ref, g0_ref, g1_ref, y_ref):
    alo, ahi = _unpack_bf16(ya_ref[...])
    blo, bhi = _unpack_bf16(yb_ref[...])
    g0 = g0_ref[...]
    g1 = g1_ref[...]
    y_ref[...] = jnp.concatenate(
        [g0 * alo + g1 * blo, g0 * ahi + g1 * bhi], axis=1)


def _wadd(yab, gates, T, D):
    WB = 1024
    NB = T // WB
    return pl.pallas_call(
        _wadd_body,
        grid=(NB,),
        in_specs=[
            pl.BlockSpec((WB, D // 2), lambda b: (b, 0)),
            pl.BlockSpec((WB, D // 2), lambda b: (NB + b, 0)),
            pl.BlockSpec((WB, 1), lambda b: (b, 0)),
            pl.BlockSpec((WB, 1), lambda b: (NB + b, 0)),
        ],
        out_specs=pl.BlockSpec((WB, D), lambda b: (b, 0)),
        out_shape=jax.ShapeDtypeStruct((T, D), jnp.float32),
    )(yab, yab, gates, gates)


def kernel(x, wg, W1, b1, W2, b2):
    T, D = x.shape
    E, _, F = W1.shape
    K = 2
    C = int(math.ceil(K * T / E * 1.25))

    sidx, gidx, gates, xpack, counts = _gate(x, wg, C, K)
    xe = _sc_dispatch(xpack, sidx, (E + 2) * C)
    ye = _ffn(counts, xe, W1, b1, W2, b2, C)
    yab = _sc_gather(ye, gidx)
    return _wadd(yab, gates, T, D)
